# baseline (speedup 1.0000x reference)
import jax
import jax.numpy as jnp
from jax import lax
from jax.experimental import pallas as pl
from jax.experimental.pallas import tpu as pltpu
from jax.experimental.pallas import tpu_sc as plsc

N = 10000
T = 4
E = 320000
D = 128
H = 128

NC = 2
NS = 16
NW = NC * NS
CHUNK = 128
CPT = 80
NBUF = 2
NGRP = CPT // NBUF
EPAD = NW * CPT * CHUNK
PAD_ROWS = 16
NP = 10240
RPT = NP // NS
DPT = (T * NP) // NS

f32 = jnp.float32
i32 = jnp.int32

BT = 512
BF = 400

_SC_MESH = plsc.VectorSubcoreMesh(core_axis_name="c", subcore_axis_name="s")


def _deg_body(idx_hbm, out_hbm, idx_v, ones_v, zb_v, deg_sh, dsem):
    c = lax.axis_index("c")
    s = lax.axis_index("s")
    wid = c * NS + s
    zeros16 = jnp.zeros((16,), f32)
    ones16 = jnp.ones((16,), f32)

    def z16(i, carry):
        zb_v[pl.ds(i * 16, 16)] = zeros16
        return carry

    lax.fori_loop(0, DPT // 16, z16, 0)

    def o16(i, carry):
        ones_v[pl.ds(i * 16, 16)] = ones16
        return carry

    lax.fori_loop(0, CHUNK // 16, o16, 0)

    off = s * DPT
    pltpu.sync_copy(zb_v, deg_sh.at[pl.ds(off, DPT)])
    plsc.subcore_barrier()

    pltpu.sync_copy(idx_hbm.at[wid], idx_v)

    DGRP = 8

    def chunk(g, carry):
        descs = [
            pltpu.async_copy(ones_v, deg_sh.at[idx_v.at[g * DGRP + b]],
                             dsem, add=True)
            for b in range(DGRP)
        ]
        for d in descs:
            d.wait()
        return carry

    lax.fori_loop(0, (T * CPT) // DGRP, chunk, 0)
    plsc.subcore_barrier()
    pltpu.sync_copy(deg_sh.at[pl.ds(off, DPT)], out_hbm.at[c, pl.ds(off, DPT)])


_deg_kernel = pl.kernel(
    _deg_body,
    out_type=jax.ShapeDtypeStruct((NC, T * NP), f32),
    mesh=_SC_MESH,
    scratch_types=[
        pltpu.VMEM((T * CPT, CHUNK), i32),
        pltpu.VMEM((CHUNK,), f32),
        pltpu.VMEM((DPT,), f32),
        pltpu.VMEM_SHARED((T * NP,), f32),
        pltpu.SemaphoreType.DMA,
    ],
)


def _seg_body(v_hbm, src_hbm, dst_hbm, out_hbm, src_stg, dst_stg, rows_v,
              acc_sh, gsem0, gsem1, ssem0, ssem1, isem0, isem1, jsem0, jsem1):
    gsems = (gsem0, gsem1)
    ssems = (ssem0, ssem1)
    isems = (isem0, isem1)
    jsems = (jsem0, jsem1)
    c = lax.axis_index("c")
    s = lax.axis_index("s")
    wid = c * NS + s
    zeros16 = jnp.zeros((16,), f32)

    def zrow(i, carry):
        def zcol(k, carry2):
            rows_v[0, i, pl.ds(k * 16, 16)] = zeros16
            return carry2

        return lax.fori_loop(0, H // 16, zcol, carry)

    lax.fori_loop(0, CHUNK, zrow, 0)

    row0 = s * RPT

    def zblk(k, carry):
        pltpu.sync_copy(rows_v.at[0], acc_sh.at[pl.ds(row0 + k * CHUNK, CHUNK)])
        return carry

    lax.fori_loop(0, RPT // CHUNK, zblk, 0)
    plsc.subcore_barrier()

    pltpu.async_copy(src_hbm.at[wid, pl.ds(0, NBUF)], src_stg.at[0], isems[0])
    pltpu.async_copy(dst_hbm.at[wid, pl.ds(0, NBUF)], dst_stg.at[0], jsems[0])
    pltpu.make_async_copy(src_hbm.at[wid, pl.ds(0, NBUF)], src_stg.at[0],
                          isems[0]).wait()
    pltpu.make_async_copy(dst_hbm.at[wid, pl.ds(0, NBUF)], dst_stg.at[0],
                          jsems[0]).wait()
    pltpu.async_copy(v_hbm.at[src_stg.at[0, 0]], rows_v.at[0], gsems[0])

    def gather(slot, b):
        pltpu.async_copy(v_hbm.at[src_stg.at[slot, b]], rows_v.at[b],
                         gsems[b])

    def gather_wait(slot, b):
        pltpu.make_async_copy(v_hbm.at[src_stg.at[slot, b]], rows_v.at[b],
                              gsems[b]).wait()

    def scatter(slot, b):
        pltpu.async_copy(rows_v.at[b], acc_sh.at[dst_stg.at[slot, b]],
                         ssems[b], add=True)

    def scatter_wait(slot, b):
        pltpu.make_async_copy(rows_v.at[b], acc_sh.at[dst_stg.at[slot, b]],
                              ssems[b]).wait()

    def outer(g2, carry):
        for p in range(2):
            g = g2 * 2 + p
            pnext = 1 - p

            gather_wait(p, 0)
            scatter(p, 0)

            @pl.when(g > 0)
            def _():
                scatter_wait(pnext, 1)

            @pl.when(g + 1 < NGRP)
            def _():
                pltpu.async_copy(
                    src_hbm.at[wid, pl.ds((g + 1) * NBUF, NBUF)],
                    src_stg.at[pnext], isems[pnext])
                pltpu.async_copy(
                    dst_hbm.at[wid, pl.ds((g + 1) * NBUF, NBUF)],
                    dst_stg.at[pnext], jsems[pnext])

            gather(p, 1)
            gather_wait(p, 1)
            scatter(p, 1)
            scatter_wait(p, 0)

            @pl.when(g + 1 < NGRP)
            def _():
                pltpu.make_async_copy(
                    src_hbm.at[wid, pl.ds((g + 1) * NBUF, NBUF)],
                    src_stg.at[pnext], isems[pnext]).wait()
                pltpu.make_async_copy(
                    dst_hbm.at[wid, pl.ds((g + 1) * NBUF, NBUF)],
                    dst_stg.at[pnext], jsems[pnext]).wait()
                gather(pnext, 0)
        return carry

    lax.fori_loop(0, NGRP // 2, outer, 0)
    scatter_wait(1, 1)
    plsc.subcore_barrier()
    pltpu.sync_copy(acc_sh.at[pl.ds(row0, RPT)], out_hbm.at[c, pl.ds(row0, RPT)])


_seg_kernel = pl.kernel(
    _seg_body,
    out_type=jax.ShapeDtypeStruct((NC, NP, H), f32),
    mesh=_SC_MESH,
    scratch_types=[
        pltpu.VMEM((2, NBUF, CHUNK), i32),
        pltpu.VMEM((2, NBUF, CHUNK), i32),
        pltpu.VMEM((NBUF, CHUNK, H), f32),
        pltpu.VMEM_SHARED((NP, H), f32),
        pltpu.SemaphoreType.DMA,
        pltpu.SemaphoreType.DMA,
        pltpu.SemaphoreType.DMA,
        pltpu.SemaphoreType.DMA,
        pltpu.SemaphoreType.DMA,
        pltpu.SemaphoreType.DMA,
        pltpu.SemaphoreType.DMA,
        pltpu.SemaphoreType.DMA,
    ],
)


def _pre0_body(x_ref, w_ref, deg_ref, o_ref):
    d = deg_ref[...]
    sc = lax.rsqrt(d[0, 0] + d[1, 0] + 1.0)
    o_ref[0] = jnp.dot(x_ref[0], w_ref[...], preferred_element_type=f32) * sc


def _pre0(x_pad, w0, degs):
    return pl.pallas_call(
        _pre0_body,
        grid=(T, NP // BT),
        in_specs=[
            pl.BlockSpec((1, BT, D), lambda t, i: (t, i, 0)),
            pl.BlockSpec((D, H), lambda t, i: (0, 0)),
            pl.BlockSpec((NC, 1, BT, 1), lambda t, i: (0, t, i, 0)),
        ],
        out_specs=pl.BlockSpec((1, BT, H), lambda t, i: (t, i, 0)),
        out_shape=jax.ShapeDtypeStruct((T, NP, H), f32),
    )(x_pad, w0, degs)


def _pre1_body(acc_ref, v_ref, deg_ref, b_ref, w_ref, o_ref):
    d = deg_ref[...]
    sc = lax.rsqrt(d[0, 0] + d[1, 0] + 1.0)
    a = acc_ref[...]
    y = jnp.maximum(sc * (a[0] + a[1] + v_ref[...]) + b_ref[...], 0.0)
    o_ref[...] = jnp.dot(y, w_ref[...], preferred_element_type=f32) * sc


def _pre1(t, acc, v, degs, b0r, w1):
    return pl.pallas_call(
        _pre1_body,
        grid=(NP // BT,),
        in_specs=[
            pl.BlockSpec((NC, BT, H), lambda i: (0, i, 0)),
            pl.BlockSpec((BT, H), lambda i: (i, 0)),
            pl.BlockSpec((NC, 1, BT, 1), lambda i, tt=t: (0, tt, i, 0)),
            pl.BlockSpec((1, H), lambda i: (0, 0)),
            pl.BlockSpec((H, H), lambda i: (0, 0)),
        ],
        out_specs=pl.BlockSpec((BT, H), lambda i: (i, 0)),
        out_shape=jax.ShapeDtypeStruct((NP, H), f32),
    )(acc, v, degs, b0r, w1)


def _fin_body(*refs):
    acc = refs[0:T]
    vv = refs[T:2 * T]
    dg = refs[2 * T:3 * T]
    b1r, wih_r, whh_r, bi_r, bh_r, o_ref = refs[3 * T:]
    bias = bi_r[...] + bh_r[...]
    h = jnp.zeros((BF, H), f32)
    cst = jnp.zeros((BF, H), f32)
    for t in range(T):
        d = dg[t][...]
        sc = lax.rsqrt(d[0, 0] + d[1, 0] + 1.0)
        a = acc[t][...]
        emb = jnp.maximum(sc * (a[0] + a[1] + vv[t][...]) + b1r[...], 0.0)
        g = (jnp.dot(emb, wih_r[...], preferred_element_type=f32)
             + jnp.dot(h, whh_r[...], preferred_element_type=f32) + bias)
        i_g = jax.nn.sigmoid(g[:, 0:H])
        f_g = jax.nn.sigmoid(g[:, H:2 * H])
        g_g = jnp.tanh(g[:, 2 * H:3 * H])
        o_g = jax.nn.sigmoid(g[:, 3 * H:4 * H])
        cst = f_g * cst + i_g * g_g
        h = o_g * jnp.tanh(cst)
    o_ref[...] = h


def _final(acc1, v1, degs, b1r, wih_t, whh_t, bir, bhr):
    in_specs = (
        [pl.BlockSpec((NC, BF, H), lambda i: (0, i, 0)) for _ in range(T)]
        + [pl.BlockSpec((BF, H), lambda i: (i, 0)) for _ in range(T)]
        + [pl.BlockSpec((NC, 1, BF, 1), lambda i, tt=t: (0, tt, i, 0))
           for t in range(T)]
        + [
            pl.BlockSpec((1, H), lambda i: (0, 0)),
            pl.BlockSpec((H, 4 * H), lambda i: (0, 0)),
            pl.BlockSpec((H, 4 * H), lambda i: (0, 0)),
            pl.BlockSpec((1, 4 * H), lambda i: (0, 0)),
            pl.BlockSpec((1, 4 * H), lambda i: (0, 0)),
        ]
    )
    return pl.pallas_call(
        _fin_body,
        grid=(N // BF,),
        in_specs=in_specs,
        out_specs=pl.BlockSpec((BF, H), lambda i: (i, 0)),
        out_shape=jax.ShapeDtypeStruct((N, H), f32),
    )(*acc1, *v1, *([degs] * T), b1r, wih_t, whh_t, bir, bhr)


def kernel(node_features_seq, edge_indices_seq, W_gcn0, b_gcn0, W_gcn1, b_gcn1,
           W_ih, W_hh, b_ih, b_hh):
    x_pad = jnp.zeros((T, NP, D), f32).at[:, :N, :].set(node_features_seq)

    pad_idx = (N + (jnp.arange(EPAD - E, dtype=i32) % PAD_ROWS))
    pad_idx = jnp.broadcast_to(pad_idx, (T, EPAD - E))
    src3 = jnp.concatenate([edge_indices_seq[:, 0, :], pad_idx], axis=1)
    dst3 = jnp.concatenate([edge_indices_seq[:, 1, :], pad_idx], axis=1)
    src3 = src3.reshape(T, NW, CPT, CHUNK)
    dst3 = dst3.reshape(T, NW, CPT, CHUNK)
    dstdeg = dst3 + (jnp.arange(T, dtype=i32) * NP)[:, None, None, None]
    dstdeg = dstdeg.transpose(1, 0, 2, 3).reshape(NW, T * CPT, CHUNK)

    deg_part = _deg_kernel(dstdeg)
    degs = deg_part.reshape(NC, T, NP, 1)

    v0 = _pre0(x_pad, W_gcn0, degs)

    b0r = b_gcn0.reshape(1, H)
    b1r = b_gcn1.reshape(1, H)
    wih_t = W_ih.T
    whh_t = W_hh.T
    bir = b_ih.reshape(1, 4 * H)
    bhr = b_hh.reshape(1, 4 * H)

    acc1_list = []
    v1_list = []
    for t in range(T):
        acc0 = _seg_kernel(v0[t], src3[t], dst3[t])
        v1 = _pre1(t, acc0, v0[t], degs, b0r, W_gcn1)
        acc1 = _seg_kernel(v1, src3[t], dst3[t])
        acc1_list.append(acc1)
        v1_list.append(v1)

    return _final(acc1_list, v1_list, degs, b1r, wih_t, whh_t, bir, bhr)


# speedup vs baseline: 1.1730x; 1.1730x over previous
import jax
import jax.numpy as jnp
from jax import lax
from jax.experimental import pallas as pl
from jax.experimental.pallas import tpu as pltpu
from jax.experimental.pallas import tpu_sc as plsc

N = 10000
T = 4
E = 320000
D = 128
H = 128

NC = 2
NS = 16
NW = NC * NS
CHUNK = 128
CPT = 80
NBUF = 2
NGRP = CPT // NBUF
EPAD = NW * CPT * CHUNK
PAD_ROWS = 16
NP = 10240
RPT = NP // NS
DPT = (T * NP) // NS

f32 = jnp.float32
i32 = jnp.int32

BT = 512
BF = 400

_SC_MESH = plsc.VectorSubcoreMesh(core_axis_name="c", subcore_axis_name="s")


def _deg_body(idx_hbm, out_hbm, idx_v, ones_v, zb_v, deg_sh, dsem):
    c = lax.axis_index("c")
    s = lax.axis_index("s")
    wid = c * NS + s
    zeros16 = jnp.zeros((16,), f32)
    ones16 = jnp.ones((16,), f32)

    def z16(i, carry):
        zb_v[pl.ds(i * 16, 16)] = zeros16
        return carry

    lax.fori_loop(0, DPT // 16, z16, 0)

    def o16(i, carry):
        ones_v[pl.ds(i * 16, 16)] = ones16
        return carry

    lax.fori_loop(0, CHUNK // 16, o16, 0)

    off = s * DPT
    pltpu.sync_copy(zb_v, deg_sh.at[pl.ds(off, DPT)])
    plsc.subcore_barrier()

    pltpu.sync_copy(idx_hbm.at[wid], idx_v)

    DGRP = 8

    def chunk(g, carry):
        descs = [
            pltpu.async_copy(ones_v, deg_sh.at[idx_v.at[g * DGRP + b]],
                             dsem, add=True)
            for b in range(DGRP)
        ]
        for d in descs:
            d.wait()
        return carry

    lax.fori_loop(0, (T * CPT) // DGRP, chunk, 0)
    plsc.subcore_barrier()
    pltpu.sync_copy(deg_sh.at[pl.ds(off, DPT)], out_hbm.at[c, pl.ds(off, DPT)])


_deg_kernel = pl.kernel(
    _deg_body,
    out_type=jax.ShapeDtypeStruct((NC, T * NP), f32),
    mesh=_SC_MESH,
    scratch_types=[
        pltpu.VMEM((T * CPT, CHUNK), i32),
        pltpu.VMEM((CHUNK,), f32),
        pltpu.VMEM((DPT,), f32),
        pltpu.VMEM_SHARED((T * NP,), f32),
        pltpu.SemaphoreType.DMA,
    ],
)


def _seg_body(v_hbm, src_hbm, dst_hbm, out_hbm, src_stg, dst_stg, rows_v,
              acc_sh, gsem0, gsem1, isem0, isem1, jsem0, jsem1):
    gsems = (gsem0, gsem1)
    isems = (isem0, isem1)
    jsems = (jsem0, jsem1)
    c = lax.axis_index("c")
    s = lax.axis_index("s")
    wid = c * NS + s
    zeros16 = jnp.zeros((16,), f32)

    def zrow(i, carry):
        def zcol(k, carry2):
            rows_v[0, i, pl.ds(k * 16, 16)] = zeros16
            return carry2

        return lax.fori_loop(0, H // 16, zcol, carry)

    lax.fori_loop(0, CHUNK, zrow, 0)

    row0 = s * RPT

    def zblk(k, carry):
        pltpu.sync_copy(rows_v.at[0], acc_sh.at[pl.ds(row0 + k * CHUNK, CHUNK)])
        return carry

    lax.fori_loop(0, RPT // CHUNK, zblk, 0)
    plsc.subcore_barrier()

    for p in range(2):
        pltpu.async_copy(src_hbm.at[wid, pl.ds(p * NBUF, NBUF)],
                         src_stg.at[p], isems[p])
        pltpu.async_copy(dst_hbm.at[wid, pl.ds(p * NBUF, NBUF)],
                         dst_stg.at[p], jsems[p])
    pltpu.make_async_copy(src_hbm.at[wid, pl.ds(0, NBUF)], src_stg.at[0],
                          isems[0]).wait()
    pltpu.make_async_copy(dst_hbm.at[wid, pl.ds(0, NBUF)], dst_stg.at[0],
                          jsems[0]).wait()
    for b in range(NBUF):
        pltpu.async_copy(v_hbm.at[src_stg.at[0, b]], rows_v.at[b], gsems[b])

    def outer(g2, carry):
        for p in range(2):
            g = g2 * 2 + p
            pnext = 1 - p

            @pl.when(g + 1 < NGRP)
            def _():
                pltpu.make_async_copy(
                    src_hbm.at[wid, pl.ds((g + 1) * NBUF, NBUF)],
                    src_stg.at[pnext], isems[pnext]).wait()
                pltpu.make_async_copy(
                    dst_hbm.at[wid, pl.ds((g + 1) * NBUF, NBUF)],
                    dst_stg.at[pnext], jsems[pnext]).wait()

            for b in range(NBUF):
                pltpu.make_async_copy(v_hbm.at[src_stg.at[p, b]],
                                      rows_v.at[b], gsems[b]).wait()
                pltpu.sync_copy(rows_v.at[b], acc_sh.at[dst_stg.at[p, b]],
                                add=True)

                @pl.when(g + 1 < NGRP)
                def _():
                    pltpu.async_copy(v_hbm.at[src_stg.at[pnext, b]],
                                     rows_v.at[b], gsems[b])

            @pl.when(g + 2 < NGRP)
            def _():
                pltpu.async_copy(src_hbm.at[wid, pl.ds((g + 2) * NBUF, NBUF)],
                                 src_stg.at[p], isems[p])
                pltpu.async_copy(dst_hbm.at[wid, pl.ds((g + 2) * NBUF, NBUF)],
                                 dst_stg.at[p], jsems[p])
        return carry

    lax.fori_loop(0, NGRP // 2, outer, 0)
    plsc.subcore_barrier()
    pltpu.sync_copy(acc_sh.at[pl.ds(row0, RPT)], out_hbm.at[c, pl.ds(row0, RPT)])


_seg_kernel = pl.kernel(
    _seg_body,
    out_type=jax.ShapeDtypeStruct((NC, NP, H), f32),
    mesh=_SC_MESH,
    scratch_types=[
        pltpu.VMEM((2, NBUF, CHUNK), i32),
        pltpu.VMEM((2, NBUF, CHUNK), i32),
        pltpu.VMEM((NBUF, CHUNK, H), f32),
        pltpu.VMEM_SHARED((NP, H), f32),
        pltpu.SemaphoreType.DMA,
        pltpu.SemaphoreType.DMA,
        pltpu.SemaphoreType.DMA,
        pltpu.SemaphoreType.DMA,
        pltpu.SemaphoreType.DMA,
        pltpu.SemaphoreType.DMA,
    ],
)


def _pre0_body(x_ref, w_ref, deg_ref, o_ref):
    d = deg_ref[...]
    sc = lax.rsqrt(d[0, 0] + d[1, 0] + 1.0)
    o_ref[0] = jnp.dot(x_ref[0], w_ref[...], preferred_element_type=f32) * sc


def _pre0(x_pad, w0, degs):
    return pl.pallas_call(
        _pre0_body,
        grid=(T, NP // BT),
        in_specs=[
            pl.BlockSpec((1, BT, D), lambda t, i: (t, i, 0)),
            pl.BlockSpec((D, H), lambda t, i: (0, 0)),
            pl.BlockSpec((NC, 1, BT, 1), lambda t, i: (0, t, i, 0)),
        ],
        out_specs=pl.BlockSpec((1, BT, H), lambda t, i: (t, i, 0)),
        out_shape=jax.ShapeDtypeStruct((T, NP, H), f32),
    )(x_pad, w0, degs)


def _pre1_body(acc_ref, v_ref, deg_ref, b_ref, w_ref, o_ref):
    d = deg_ref[...]
    sc = lax.rsqrt(d[0, 0] + d[1, 0] + 1.0)
    a = acc_ref[...]
    y = jnp.maximum(sc * (a[0] + a[1] + v_ref[...]) + b_ref[...], 0.0)
    o_ref[...] = jnp.dot(y, w_ref[...], preferred_element_type=f32) * sc


def _pre1(t, acc, v, degs, b0r, w1):
    return pl.pallas_call(
        _pre1_body,
        grid=(NP // BT,),
        in_specs=[
            pl.BlockSpec((NC, BT, H), lambda i: (0, i, 0)),
            pl.BlockSpec((BT, H), lambda i: (i, 0)),
            pl.BlockSpec((NC, 1, BT, 1), lambda i, tt=t: (0, tt, i, 0)),
            pl.BlockSpec((1, H), lambda i: (0, 0)),
            pl.BlockSpec((H, H), lambda i: (0, 0)),
        ],
        out_specs=pl.BlockSpec((BT, H), lambda i: (i, 0)),
        out_shape=jax.ShapeDtypeStruct((NP, H), f32),
    )(acc, v, degs, b0r, w1)


def _fin_body(*refs):
    acc = refs[0:T]
    vv = refs[T:2 * T]
    dg = refs[2 * T:3 * T]
    b1r, wih_r, whh_r, bi_r, bh_r, o_ref = refs[3 * T:]
    bias = bi_r[...] + bh_r[...]
    h = jnp.zeros((BF, H), f32)
    cst = jnp.zeros((BF, H), f32)
    for t in range(T):
        d = dg[t][...]
        sc = lax.rsqrt(d[0, 0] + d[1, 0] + 1.0)
        a = acc[t][...]
        emb = jnp.maximum(sc * (a[0] + a[1] + vv[t][...]) + b1r[...], 0.0)
        g = (jnp.dot(emb, wih_r[...], preferred_element_type=f32)
             + jnp.dot(h, whh_r[...], preferred_element_type=f32) + bias)
        i_g = jax.nn.sigmoid(g[:, 0:H])
        f_g = jax.nn.sigmoid(g[:, H:2 * H])
        g_g = jnp.tanh(g[:, 2 * H:3 * H])
        o_g = jax.nn.sigmoid(g[:, 3 * H:4 * H])
        cst = f_g * cst + i_g * g_g
        h = o_g * jnp.tanh(cst)
    o_ref[...] = h


def _final(acc1, v1, degs, b1r, wih_t, whh_t, bir, bhr):
    in_specs = (
        [pl.BlockSpec((NC, BF, H), lambda i: (0, i, 0)) for _ in range(T)]
        + [pl.BlockSpec((BF, H), lambda i: (i, 0)) for _ in range(T)]
        + [pl.BlockSpec((NC, 1, BF, 1), lambda i, tt=t: (0, tt, i, 0))
           for t in range(T)]
        + [
            pl.BlockSpec((1, H), lambda i: (0, 0)),
            pl.BlockSpec((H, 4 * H), lambda i: (0, 0)),
            pl.BlockSpec((H, 4 * H), lambda i: (0, 0)),
            pl.BlockSpec((1, 4 * H), lambda i: (0, 0)),
            pl.BlockSpec((1, 4 * H), lambda i: (0, 0)),
        ]
    )
    return pl.pallas_call(
        _fin_body,
        grid=(N // BF,),
        in_specs=in_specs,
        out_specs=pl.BlockSpec((BF, H), lambda i: (i, 0)),
        out_shape=jax.ShapeDtypeStruct((N, H), f32),
    )(*acc1, *v1, *([degs] * T), b1r, wih_t, whh_t, bir, bhr)


def kernel(node_features_seq, edge_indices_seq, W_gcn0, b_gcn0, W_gcn1, b_gcn1,
           W_ih, W_hh, b_ih, b_hh):
    x_pad = jnp.zeros((T, NP, D), f32).at[:, :N, :].set(node_features_seq)

    pad_idx = (N + (jnp.arange(EPAD - E, dtype=i32) % PAD_ROWS))
    pad_idx = jnp.broadcast_to(pad_idx, (T, EPAD - E))
    src3 = jnp.concatenate([edge_indices_seq[:, 0, :], pad_idx], axis=1)
    dst3 = jnp.concatenate([edge_indices_seq[:, 1, :], pad_idx], axis=1)
    src3 = src3.reshape(T, NW, CPT, CHUNK)
    dst3 = dst3.reshape(T, NW, CPT, CHUNK)
    dstdeg = dst3 + (jnp.arange(T, dtype=i32) * NP)[:, None, None, None]
    dstdeg = dstdeg.transpose(1, 0, 2, 3).reshape(NW, T * CPT, CHUNK)

    deg_part = _deg_kernel(dstdeg)
    degs = deg_part.reshape(NC, T, NP, 1)

    v0 = _pre0(x_pad, W_gcn0, degs)

    b0r = b_gcn0.reshape(1, H)
    b1r = b_gcn1.reshape(1, H)
    wih_t = W_ih.T
    whh_t = W_hh.T
    bir = b_ih.reshape(1, 4 * H)
    bhr = b_hh.reshape(1, 4 * H)

    acc1_list = []
    v1_list = []
    for t in range(T):
        acc0 = _seg_kernel(v0[t], src3[t], dst3[t])
        v1 = _pre1(t, acc0, v0[t], degs, b0r, W_gcn1)
        acc1 = _seg_kernel(v1, src3[t], dst3[t])
        acc1_list.append(acc1)
        v1_list.append(v1)

    return _final(acc1_list, v1_list, degs, b1r, wih_t, whh_t, bir, bhr)
